# baseline (speedup 1.0000x reference)
import functools

import jax
import jax.numpy as jnp
from jax import lax
from jax.experimental import pallas as pl
from jax.experimental.pallas import tpu as pltpu

N_ROWS = 16384
N_COLS = 1000
BLOCK_ROWS = 256
K = max(1, N_ROWS * 10 // 100)


def _ce_block_kernel(x_ref, t_ref, ce_ref):
    x = x_ref[...]
    t = t_ref[...]
    m = jnp.max(x, axis=1, keepdims=True)
    s = jnp.sum(jnp.exp(x - m), axis=1, keepdims=True)
    col = lax.broadcasted_iota(jnp.int32, x.shape, 1)
    picked = jnp.sum(jnp.where(col == t, x, 0.0), axis=1, keepdims=True)
    ce_ref[...] = (m - picked) + jnp.log(s)


def _topk_mean_kernel(ce_ref, out_ref):
    ce = ce_ref[...]
    bits = lax.bitcast_convert_type(ce, jnp.int32)
    kf = jnp.float32(K)

    def body(_, carry):
        lo, hi = carry
        mid = lo + (hi - lo) // 2
        cnt = jnp.sum((bits >= mid).astype(jnp.int32))
        ge = cnt >= K
        return jnp.where(ge, mid, lo), jnp.where(ge, hi, mid)

    lo0 = jnp.int32(-1)
    hi0 = jnp.int32(0x7F800001)
    lo, _ = lax.fori_loop(0, 31, body, (lo0, hi0))

    gt = bits > lo
    cnt_gt = jnp.sum(gt.astype(jnp.float32))
    sum_gt = jnp.sum(jnp.where(gt, ce, 0.0))
    kth = jnp.max(jnp.where(bits == lo, ce, 0.0))
    out_ref[0, 0] = (sum_gt + kth * (kf - cnt_gt)) / kf


@functools.partial(jax.jit)
def kernel(inputs, targets):
    t2d = targets.astype(jnp.int32).reshape(N_ROWS, 1)
    grid = N_ROWS // BLOCK_ROWS
    ce = pl.pallas_call(
        _ce_block_kernel,
        grid=(grid,),
        in_specs=[
            pl.BlockSpec((BLOCK_ROWS, N_COLS), lambda b: (b, 0)),
            pl.BlockSpec((BLOCK_ROWS, 1), lambda b: (b, 0)),
        ],
        out_specs=pl.BlockSpec((BLOCK_ROWS, 1), lambda b: (b, 0)),
        out_shape=jax.ShapeDtypeStruct((N_ROWS, 1), jnp.float32),
    )(inputs, t2d)

    ce2d = ce.reshape(128, 128)
    out = pl.pallas_call(
        _topk_mean_kernel,
        out_specs=pl.BlockSpec(memory_space=pltpu.SMEM),
        out_shape=jax.ShapeDtypeStruct((1, 1), jnp.float32),
    )(ce2d)
    return out.reshape(())


# speedup vs baseline: 1.2816x; 1.2816x over previous
import functools

import jax
import jax.numpy as jnp
from jax import lax
from jax.experimental import pallas as pl
from jax.experimental.pallas import tpu as pltpu

N_ROWS = 16384
N_COLS = 1000
BLOCK_ROWS = 512
GRID = N_ROWS // BLOCK_ROWS
K = max(1, N_ROWS * 10 // 100)


def _fused_kernel(x_ref, t_ref, out_ref, ce_s):
    b = pl.program_id(0)
    x = x_ref[...]
    t = t_ref[0]
    tcol = t.reshape(BLOCK_ROWS, 1)
    m = jnp.max(x, axis=1, keepdims=True)
    s = jnp.sum(jnp.exp(x - m), axis=1, keepdims=True)
    col = lax.broadcasted_iota(jnp.int32, x.shape, 1)
    picked = jnp.sum(jnp.where(col == tcol, x, 0.0), axis=1, keepdims=True)
    ce = (m - picked) + jnp.log(s)
    ce_s[pl.ds(b, 1), :] = ce.reshape(1, BLOCK_ROWS)

    @pl.when(b == GRID - 1)
    def _select():
        cev = ce_s[...]
        bits = lax.bitcast_convert_type(cev, jnp.int32)
        kf = jnp.float32(K)

        def body(_, carry):
            lo, hi = carry
            mid = lo + (hi - lo) // 2
            cnt = jnp.sum((bits >= mid).astype(jnp.int32))
            ge = cnt >= K
            return jnp.where(ge, mid, lo), jnp.where(ge, hi, mid)

        lo0 = jnp.int32(-1)
        hi0 = jnp.int32(0x7F800001)
        lo, _ = lax.fori_loop(0, 31, body, (lo0, hi0))

        gt = bits > lo
        cnt_gt = jnp.sum(gt.astype(jnp.float32))
        sum_gt = jnp.sum(jnp.where(gt, cev, 0.0))
        kth = jnp.max(jnp.where(bits == lo, cev, 0.0))
        out_ref[0, 0] = (sum_gt + kth * (kf - cnt_gt)) / kf


@functools.partial(jax.jit)
def kernel(inputs, targets):
    t3d = targets.astype(jnp.int32).reshape(GRID, 1, BLOCK_ROWS)
    out = pl.pallas_call(
        _fused_kernel,
        grid=(GRID,),
        in_specs=[
            pl.BlockSpec((BLOCK_ROWS, N_COLS), lambda b: (b, 0)),
            pl.BlockSpec((1, 1, BLOCK_ROWS), lambda b: (b, 0, 0)),
        ],
        out_specs=pl.BlockSpec(memory_space=pltpu.SMEM),
        out_shape=jax.ShapeDtypeStruct((1, 1), jnp.float32),
        scratch_shapes=[pltpu.VMEM((GRID, BLOCK_ROWS), jnp.float32)],
    )(inputs, t3d)
    return out.reshape(())
